# baseline (speedup 1.0000x reference)
import functools

import jax
import jax.numpy as jnp
from jax import lax
from jax.experimental import pallas as pl
from jax.experimental.pallas import tpu as pltpu
from jax.experimental.pallas import tpu_sc as plsc

_NUM_CODES = 8192
_CODE_DIM = 32
_BETA = 0.25
_BZ = 1024
_BE = 2048

_SC_CORES = 2
_SC_SUBCORES = 16
_SC_WORKERS = _SC_CORES * _SC_SUBCORES
_SC_CHUNK = 128


def _dist_argmin_body(z_ref, e_ref, z2_ref, e2_ref, idx_ref, loss_ref,
                      best_val, best_losf, best_idx, acc):
    i = pl.program_id(0)
    j = pl.program_id(1)
    mm = lax.dot_general(z_ref[...], e_ref[...], (((1,), (1,)), ((), ())),
                         preferred_element_type=jnp.float32)
    dist = (z2_ref[...].reshape(_BZ, 1) + e2_ref[...].reshape(1, _BE)) - 2.0 * mm
    lmin = jnp.min(dist, axis=1)
    iota = lax.broadcasted_iota(jnp.int32, (_BZ, _BE), 1) + j * _BE
    larg = jnp.min(jnp.where(dist == lmin[:, None], iota, jnp.int32(2**30)),
                   axis=1)

    @pl.when(j == 0)
    def _():
        best_val[...] = lmin.astype(jnp.bfloat16)
        best_losf[...] = lmin
        best_idx[...] = larg

    @pl.when(j != 0)
    def _():
        bv = best_val[...].astype(jnp.float32)
        upd = lmin < bv
        best_val[...] = jnp.where(upd, lmin, bv).astype(jnp.bfloat16)
        best_losf[...] = jnp.where(upd, lmin, best_losf[...])
        best_idx[...] = jnp.where(upd, larg, best_idx[...])

    @pl.when(j == pl.num_programs(1) - 1)
    def _():
        idx_ref[...] = best_idx[...]

        @pl.when(i == 0)
        def _():
            acc[0] = 0.0

        acc[0] += jnp.sum(best_losf[...])

        @pl.when(i == pl.num_programs(0) - 1)
        def _():
            loss_ref[0] = acc[0]


def _dist_argmin(z_bf, e_bf, z2, e2):
    n = z_bf.shape[0]
    grid = (n // _BZ, _NUM_CODES // _BE)
    return pl.pallas_call(
        _dist_argmin_body,
        grid=grid,
        in_specs=[
            pl.BlockSpec((_BZ, _CODE_DIM), lambda i, j: (i, 0)),
            pl.BlockSpec((_BE, _CODE_DIM), lambda i, j: (j, 0)),
            pl.BlockSpec((_BZ,), lambda i, j: (i,)),
            pl.BlockSpec((_BE,), lambda i, j: (j,)),
        ],
        out_specs=[
            pl.BlockSpec((_BZ,), lambda i, j: (i,)),
            pl.BlockSpec(memory_space=pltpu.SMEM),
        ],
        out_shape=[
            jax.ShapeDtypeStruct((n,), jnp.int32),
            jax.ShapeDtypeStruct((1,), jnp.float32),
        ],
        scratch_shapes=[
            pltpu.VMEM((_BZ,), jnp.bfloat16),
            pltpu.VMEM((_BZ,), jnp.float32),
            pltpu.VMEM((_BZ,), jnp.int32),
            pltpu.SMEM((1,), jnp.float32),
        ],
    )(z_bf, e_bf, z2, e2)


def _make_sc_gather(n_rows):
    b_per_w = n_rows // _SC_WORKERS
    mesh = plsc.VectorSubcoreMesh(core_axis_name="c", subcore_axis_name="s")

    @functools.partial(
        pl.kernel,
        mesh=mesh,
        out_type=jax.ShapeDtypeStruct((n_rows, _CODE_DIM), jnp.float32),
        scratch_types=[
            pltpu.VMEM((b_per_w,), jnp.int32),
            pltpu.VMEM((b_per_w, _CODE_DIM), jnp.float32),
            pltpu.SemaphoreType.DMA,
        ],
        compiler_params=pltpu.CompilerParams(use_tc_tiling_on_sc=False),
    )
    def gather(table_hbm, idx_hbm, out_hbm, idx_v, rows_v, sem):
        wid = lax.axis_index("s") * _SC_CORES + lax.axis_index("c")
        base = wid * b_per_w
        pltpu.sync_copy(idx_hbm.at[pl.ds(base, b_per_w)], idx_v)
        for c in range(b_per_w // _SC_CHUNK):
            pltpu.async_copy(
                table_hbm.at[idx_v.at[pl.ds(c * _SC_CHUNK, _SC_CHUNK)]],
                rows_v.at[pl.ds(c * _SC_CHUNK, _SC_CHUNK)],
                sem,
            ).wait()
        pltpu.sync_copy(rows_v, out_hbm.at[pl.ds(base, b_per_w)])

    return gather


def kernel(z_e, codebook):
    B, D, H, W = z_e.shape
    z_flat = jnp.transpose(z_e, (0, 2, 3, 1)).reshape(-1, D)
    z2 = (z_flat ** 2).sum(axis=1)
    e2 = (codebook ** 2).sum(axis=1)
    z_bf = z_flat.astype(jnp.bfloat16)
    e_bf = codebook.astype(jnp.bfloat16)
    idx_flat, loss_sum = _dist_argmin(z_bf, e_bf, z2, e2)
    z_q_flat = _make_sc_gather(z_flat.shape[0])(codebook, idx_flat)
    z_q = z_q_flat.reshape(B, H, W, D).transpose(0, 3, 1, 2)
    loss = loss_sum[0] / z_e.size
    vq_loss = loss + _BETA * loss
    z_q_st = z_e + lax.stop_gradient(z_q - z_e)
    indices_map = idx_flat.reshape(B, H, W)
    return (z_q_st, indices_map, vq_loss)


# speedup vs baseline: 1.2424x; 1.2424x over previous
import functools

import jax
import jax.numpy as jnp
from jax import lax
from jax.experimental import pallas as pl
from jax.experimental.pallas import tpu as pltpu
from jax.experimental.pallas import tpu_sc as plsc

_NUM_CODES = 8192
_CODE_DIM = 32
_BETA = 0.25
_BZ = 1024
_BE = 2048

_SC_CORES = 2
_SC_SUBCORES = 16
_SC_WORKERS = _SC_CORES * _SC_SUBCORES
_SC_CHUNK = 128


def _dist_argmin_body(z_ref, e_ref, z2_ref, e2_ref, idx_ref, loss_ref,
                      best_val, best_losf, best_idx, acc):
    i = pl.program_id(0)
    j = pl.program_id(1)
    mm = lax.dot_general(e_ref[...], z_ref[...], (((1,), (1,)), ((), ())),
                         preferred_element_type=jnp.float32)
    dist = (e2_ref[...].reshape(_BE, 1) + z2_ref[...].reshape(1, _BZ)) - 2.0 * mm
    lmin = jnp.min(dist, axis=0)
    iota = lax.broadcasted_iota(jnp.int32, (_BE, _BZ), 0) + j * _BE
    larg = jnp.min(jnp.where(dist == lmin[None, :], iota, jnp.int32(2**30)),
                   axis=0)

    @pl.when(j == 0)
    def _():
        best_val[...] = lmin.astype(jnp.bfloat16)
        best_losf[...] = lmin
        best_idx[...] = larg

    @pl.when(j != 0)
    def _():
        bv = best_val[...].astype(jnp.float32)
        upd = lmin < bv
        best_val[...] = jnp.where(upd, lmin, bv).astype(jnp.bfloat16)
        best_losf[...] = jnp.where(upd, lmin, best_losf[...])
        best_idx[...] = jnp.where(upd, larg, best_idx[...])

    @pl.when(j == pl.num_programs(1) - 1)
    def _():
        idx_ref[...] = best_idx[...]

        @pl.when(i == 0)
        def _():
            acc[0] = 0.0

        acc[0] += jnp.sum(best_losf[...])

        @pl.when(i == pl.num_programs(0) - 1)
        def _():
            loss_ref[0] = acc[0]


def _dist_argmin(z_bf, e_bf, z2, e2):
    n = z_bf.shape[0]
    grid = (n // _BZ, _NUM_CODES // _BE)
    return pl.pallas_call(
        _dist_argmin_body,
        grid=grid,
        in_specs=[
            pl.BlockSpec((_BZ, _CODE_DIM), lambda i, j: (i, 0)),
            pl.BlockSpec((_BE, _CODE_DIM), lambda i, j: (j, 0)),
            pl.BlockSpec((_BZ,), lambda i, j: (i,)),
            pl.BlockSpec((_BE,), lambda i, j: (j,)),
        ],
        out_specs=[
            pl.BlockSpec((_BZ,), lambda i, j: (i,)),
            pl.BlockSpec(memory_space=pltpu.SMEM),
        ],
        out_shape=[
            jax.ShapeDtypeStruct((n,), jnp.int32),
            jax.ShapeDtypeStruct((1,), jnp.float32),
        ],
        scratch_shapes=[
            pltpu.VMEM((_BZ,), jnp.bfloat16),
            pltpu.VMEM((_BZ,), jnp.float32),
            pltpu.VMEM((_BZ,), jnp.int32),
            pltpu.SMEM((1,), jnp.float32),
        ],
    )(z_bf, e_bf, z2, e2)


def _make_sc_gather(n_rows):
    b_per_w = n_rows // _SC_WORKERS
    mesh = plsc.VectorSubcoreMesh(core_axis_name="c", subcore_axis_name="s")

    @functools.partial(
        pl.kernel,
        mesh=mesh,
        out_type=jax.ShapeDtypeStruct((n_rows, _CODE_DIM), jnp.float32),
        scratch_types=[
            pltpu.VMEM((b_per_w,), jnp.int32),
            pltpu.VMEM((b_per_w, _CODE_DIM), jnp.float32),
            pltpu.SemaphoreType.DMA,
        ],
        compiler_params=pltpu.CompilerParams(use_tc_tiling_on_sc=False),
    )
    def gather(table_hbm, idx_hbm, out_hbm, idx_v, rows_v, sem):
        wid = lax.axis_index("s") * _SC_CORES + lax.axis_index("c")
        base = wid * b_per_w
        pltpu.sync_copy(idx_hbm.at[pl.ds(base, b_per_w)], idx_v)
        for c in range(b_per_w // _SC_CHUNK):
            pltpu.async_copy(
                table_hbm.at[idx_v.at[pl.ds(c * _SC_CHUNK, _SC_CHUNK)]],
                rows_v.at[pl.ds(c * _SC_CHUNK, _SC_CHUNK)],
                sem,
            ).wait()
        pltpu.sync_copy(rows_v, out_hbm.at[pl.ds(base, b_per_w)])

    return gather


def kernel(z_e, codebook):
    B, D, H, W = z_e.shape
    z_flat = jnp.transpose(z_e, (0, 2, 3, 1)).reshape(-1, D)
    z2 = (z_flat ** 2).sum(axis=1)
    e2 = (codebook ** 2).sum(axis=1)
    z_bf = z_flat.astype(jnp.bfloat16)
    e_bf = codebook.astype(jnp.bfloat16)
    idx_flat, loss_sum = _dist_argmin(z_bf, e_bf, z2, e2)
    z_q_flat = _make_sc_gather(z_flat.shape[0])(codebook, idx_flat)
    z_q = z_q_flat.reshape(B, H, W, D).transpose(0, 3, 1, 2)
    loss = loss_sum[0] / z_e.size
    vq_loss = loss + _BETA * loss
    z_q_st = z_e + lax.stop_gradient(z_q - z_e)
    indices_map = idx_flat.reshape(B, H, W)
    return (z_q_st, indices_map, vq_loss)
